# baseline (speedup 1.0000x reference)
import functools

import jax
import jax.numpy as jnp
from jax import lax
from jax.experimental import pallas as pl
from jax.experimental.pallas import tpu as pltpu
from jax.experimental.pallas import tpu_sc as plsc


def _rowsum_body(x_ref, o_ref):
    o_ref[...] = jnp.sum(jnp.log(x_ref[...]), axis=1, keepdims=True)


def _combine_body(s_ref, g_ref, o_ref, *, ncols):
    o_ref[...] = (s_ref[...] - jnp.log(g_ref[...])) * (-1.0 / ncols)


def _sc_gather(pred_flat, flat_idx):
    B = flat_idx.shape[0]
    info = plsc.get_sparse_core_info()
    nw = info.num_cores * info.num_subcores
    bpw = B // nw
    mesh = plsc.VectorSubcoreMesh(core_axis_name="c", subcore_axis_name="s")

    @functools.partial(
        pl.kernel,
        out_type=jax.ShapeDtypeStruct((B,), jnp.float32),
        mesh=mesh,
        scratch_types=[
            pltpu.VMEM((bpw,), jnp.int32),
            pltpu.VMEM((bpw,), jnp.float32),
            pltpu.SemaphoreType.DMA,
        ],
    )
    def gk(table_hbm, idx_hbm, out_hbm, idx_v, vals_v, sem):
        wid = lax.axis_index("s") * info.num_cores + lax.axis_index("c")
        base = wid * bpw
        pltpu.sync_copy(idx_hbm.at[pl.ds(base, bpw)], idx_v)
        pltpu.async_copy(table_hbm.at[idx_v], vals_v, sem).wait()
        pltpu.sync_copy(vals_v, out_hbm.at[pl.ds(base, bpw)])

    return gk(pred_flat, flat_idx)


def kernel(pred, target):
    B, C = pred.shape
    BR = 16
    t32 = target.astype(jnp.int32)
    flat_idx = jnp.arange(B, dtype=jnp.int32) * C + t32
    gathered = _sc_gather(pred.reshape(-1), flat_idx)

    rowsums = pl.pallas_call(
        _rowsum_body,
        grid=(B // BR,),
        in_specs=[pl.BlockSpec((BR, C), lambda i: (i, 0))],
        out_specs=pl.BlockSpec((BR, 1), lambda i: (i, 0)),
        out_shape=jax.ShapeDtypeStruct((B, 1), jnp.float32),
        compiler_params=pltpu.CompilerParams(
            dimension_semantics=("parallel",)),
    )(pred)

    out = pl.pallas_call(
        functools.partial(_combine_body, ncols=C),
        in_specs=[
            pl.BlockSpec((B, 1), lambda: (0, 0)),
            pl.BlockSpec((B, 1), lambda: (0, 0)),
        ],
        out_specs=pl.BlockSpec((B, 1), lambda: (0, 0)),
        out_shape=jax.ShapeDtypeStruct((B, 1), jnp.float32),
    )(rowsums, gathered.reshape(B, 1))
    return out[:, 0]


# speedup vs baseline: 2.2418x; 2.2418x over previous
import functools
import math

import jax
import jax.numpy as jnp
from jax.experimental import pallas as pl


def _loss_body(t_ref, x_ref, o_ref, *, bc, ncols, nblk):
    j = pl.program_id(0)
    rows = x_ref.shape[0]
    cols = jax.lax.broadcasted_iota(jnp.int32, (rows, bc), 1)
    t_loc = t_ref[...] - j * bc

    def accum(s):
        @pl.when(j == 0)
        def _():
            o_ref[...] = s

        @pl.when(j > 0)
        def _():
            o_ref[...] += s

    @pl.when(j < nblk - 1)
    def _main():
        logs = jnp.log2(x_ref[...])
        accum(jnp.sum(jnp.where(cols == t_loc, 0.0, logs),
                      axis=1, keepdims=True))

    @pl.when(j == nblk - 1)
    def _last():
        nvalid = ncols - (nblk - 1) * bc
        logs = jnp.log2(x_ref[...])
        accum(jnp.sum(jnp.where((cols == t_loc) | (cols >= nvalid), 0.0, logs),
                      axis=1, keepdims=True))
        o_ref[...] = o_ref[...] * (-math.log(2.0) / ncols)


def kernel(pred, target):
    B, C = pred.shape
    BC = 4096
    nblk = pl.cdiv(C, BC)
    t2 = target.astype(jnp.int32).reshape(B, 1)
    out = pl.pallas_call(
        functools.partial(_loss_body, bc=BC, ncols=C, nblk=nblk),
        grid=(nblk,),
        in_specs=[
            pl.BlockSpec((B, 1), lambda j: (0, 0)),
            pl.BlockSpec((B, BC), lambda j: (0, j)),
        ],
        out_specs=pl.BlockSpec((B, 1), lambda j: (0, 0)),
        out_shape=jax.ShapeDtypeStruct((B, 1), jnp.float32),
    )(t2, pred)
    return out[:, 0]


# speedup vs baseline: 2.2422x; 1.0002x over previous
import functools
import math

import jax
import jax.numpy as jnp
from jax import lax
from jax.experimental import pallas as pl
from jax.experimental.pallas import tpu as pltpu

_BC = 2048
_NBUF = 4


def _body(x_hbm, t_ref, o_ref, buf, tbuf, sems, tsem, *, ncols):
    B = t_ref.shape[0]
    nfull = ncols // _BC
    tail = ncols - nfull * _BC
    cols = lax.broadcasted_iota(jnp.int32, (B, _BC), 1)
    tgt = t_ref[...]

    if tail:
        pltpu.make_async_copy(
            x_hbm.at[:, pl.ds(nfull * _BC, tail)], tbuf, tsem,
        ).start()

    def issue(k, slot):
        pltpu.make_async_copy(
            x_hbm.at[:, pl.ds(k * _BC, _BC)],
            buf.at[slot],
            sems.at[slot],
        ).start()

    def wait(slot):
        pltpu.make_async_copy(
            x_hbm.at[:, pl.ds(0, _BC)],
            buf.at[slot],
            sems.at[slot],
        ).wait()

    def block_sum(slot, k):
        logs = jnp.log2(buf[slot])
        return jnp.sum(jnp.where(cols + k * _BC == tgt, 0.0, logs),
                       axis=1, keepdims=True)

    for k in range(min(_NBUF, nfull)):
        issue(k, k)

    nquad = nfull // _NBUF

    def quad(q, carry):
        k0 = q * _NBUF
        acc = carry
        for s in range(_NBUF):
            wait(s)
            acc = acc + block_sum(s, k0 + s)
            nxt = k0 + _NBUF + s

            @pl.when(nxt < nfull)
            def _():
                issue(nxt, s)
        return acc

    acc = lax.fori_loop(0, nquad, quad, jnp.zeros((B, 1), jnp.float32))

    for k in range(nquad * _NBUF, nfull):
        s = k % _NBUF
        wait(s)
        acc = acc + block_sum(s, k)

    if tail:
        pltpu.make_async_copy(
            x_hbm.at[:, pl.ds(nfull * _BC, tail)], tbuf, tsem,
        ).wait()
        logs = jnp.log2(tbuf[...])
        tcols = lax.broadcasted_iota(jnp.int32, (B, tail), 1) + nfull * _BC
        acc = acc + jnp.sum(jnp.where(tcols == tgt, 0.0, logs),
                            axis=1, keepdims=True)

    o_ref[...] = acc * (-math.log(2.0) / ncols)


def kernel(pred, target):
    B, C = pred.shape
    t2 = target.astype(jnp.int32).reshape(B, 1)
    out = pl.pallas_call(
        functools.partial(_body, ncols=C),
        in_specs=[
            pl.BlockSpec(memory_space=pltpu.MemorySpace.HBM),
            pl.BlockSpec((B, 1), lambda: (0, 0)),
        ],
        out_specs=pl.BlockSpec((B, 1), lambda: (0, 0)),
        out_shape=jax.ShapeDtypeStruct((B, 1), jnp.float32),
        scratch_shapes=[
            pltpu.VMEM((_NBUF, B, _BC), jnp.float32),
            pltpu.VMEM((B, C - (C // _BC) * _BC), jnp.float32),
            pltpu.SemaphoreType.DMA((_NBUF,)),
            pltpu.SemaphoreType.DMA,
        ],
    )(pred, t2)
    return out[:, 0]
